# baseline (speedup 1.0000x reference)
import jax
import jax.numpy as jnp
from jax import lax
from jax.experimental import pallas as pl
from jax.experimental.pallas import tpu as pltpu
from jax.experimental.pallas import tpu_sc as plsc

MU = 3.5
B = 16384
F = 32
NC, NS, L = 2, 16, 16
NW = NC * NS
RPW = B // NW
CHUNK = 128
NCHUNK = RPW // CHUNK
HPAD = 17


def _svdpp_body(user_hbm, item_hbm, P_hbm, Q_hbm, bu_hbm, bi_hbm, out_hbm,
                uidx, iidx, pu, qi, buv, biv, hbuf, outv, sem):
    wid = lax.axis_index("s") * NC + lax.axis_index("c")
    base = wid * RPW

    pltpu.sync_copy(user_hbm.at[pl.ds(wid * NCHUNK, NCHUNK)], uidx)
    pltpu.sync_copy(item_hbm.at[pl.ds(wid * NCHUNK, NCHUNK)], iidx)

    copies = []
    for j in range(NCHUNK):
        sl = pl.ds(j * CHUNK, CHUNK)
        copies.append(pltpu.async_copy(P_hbm.at[uidx.at[j]], pu.at[sl], sem))
        copies.append(pltpu.async_copy(Q_hbm.at[iidx.at[j]], qi.at[sl], sem))
        copies.append(pltpu.async_copy(bu_hbm.at[uidx.at[j]], buv.at[sl], sem))
        copies.append(pltpu.async_copy(bi_hbm.at[iidx.at[j]], biv.at[sl], sem))
    for c in copies:
        c.wait()

    lanes = lax.iota(jnp.int32, L)

    def s1(b, carry):
        p0 = pu[b, pl.ds(0, L)]
        p1 = pu[b, pl.ds(L, L)]
        q0 = qi[b, pl.ds(0, L)]
        q1 = qi[b, pl.ds(L, L)]
        plsc.store_scatter(hbuf, [b * HPAD + lanes], p0 * q0 + p1 * q1)
        return carry

    lax.fori_loop(0, RPW, s1, 0)

    def s2(i, carry):
        rvec = i * L + lanes
        acc = plsc.load_gather(buv, [rvec]) + plsc.load_gather(biv, [rvec]) + MU
        hbase = rvec * HPAD
        for j in range(L):
            acc = acc + plsc.load_gather(hbuf, [hbase + j])
        plsc.store_scatter(outv, [rvec], acc)
        return carry

    lax.fori_loop(0, RPW // L, s2, 0)

    pltpu.sync_copy(outv, out_hbm.at[pl.ds(base, RPW)])


def kernel(x, P, Q, bu, bi):
    user = x[:, 0].astype(jnp.int32).reshape(B // CHUNK, CHUNK)
    item = x[:, 1].astype(jnp.int32).reshape(B // CHUNK, CHUNK)
    mesh = plsc.VectorSubcoreMesh(core_axis_name="c", subcore_axis_name="s")
    k = pl.kernel(
        _svdpp_body,
        out_type=jax.ShapeDtypeStruct((B,), jnp.float32),
        mesh=mesh,
        compiler_params=pltpu.CompilerParams(
            needs_layout_passes=False, use_tc_tiling_on_sc=False),
        scratch_types=[
            pltpu.VMEM((NCHUNK, CHUNK), jnp.int32),
            pltpu.VMEM((NCHUNK, CHUNK), jnp.int32),
            pltpu.VMEM((RPW, F), jnp.float32),
            pltpu.VMEM((RPW, F), jnp.float32),
            pltpu.VMEM((RPW,), jnp.float32),
            pltpu.VMEM((RPW,), jnp.float32),
            pltpu.VMEM((RPW * HPAD,), jnp.float32),
            pltpu.VMEM((RPW,), jnp.float32),
            pltpu.SemaphoreType.DMA,
        ],
    )
    return k(user, item, P, Q, bu[:, 0], bi[:, 0])
